# baseline (speedup 1.0000x reference)
import jax
import jax.numpy as jnp
from jax import lax
from jax.experimental import pallas as pl
from jax.experimental.pallas import tpu as pltpu
from jax.experimental.pallas import tpu_sc as plsc

NUM_CORES = 2
NUM_SUBCORES = 16
NUM_WORKERS = NUM_CORES * NUM_SUBCORES

CHUNK = 320
NBUF = 4


def _transpose_kernel(d, v):
    n_full = v // 128
    per_w = n_full // NUM_WORKERS
    extra = n_full % NUM_WORKERS
    rpb = d
    mesh = plsc.VectorSubcoreMesh(core_axis_name="c", subcore_axis_name="s")

    def body(tt_hbm, out_hbm, blk_v, trans_v, sem_i, sem_o):
        iota = lax.iota(jnp.int32, 16)
        wid = lax.axis_index("s") * NUM_CORES + lax.axis_index("c")
        base = wid * per_w + jnp.minimum(wid, extra)
        cnt = per_w + jnp.where(wid < extra, 1, 0)

        def start_in(c, slot):
            pltpu.async_copy(tt_hbm.at[:, pl.ds(c * 128, 128)],
                             blk_v.at[slot], sem_i.at[slot])

        def wait_in(slot):
            pltpu.make_async_copy(tt_hbm.at[:, pl.ds(0, 128)],
                                  blk_v.at[slot], sem_i.at[slot]).wait()

        def start_out(c, slot):
            pltpu.async_copy(trans_v.at[slot],
                             out_hbm.at[pl.ds(c * rpb, rpb)], sem_o.at[slot])

        def wait_out(slot):
            pltpu.make_async_copy(trans_v.at[slot],
                                  out_hbm.at[pl.ds(0, rpb)],
                                  sem_o.at[slot]).wait()

        def transpose_block(slot):
            def jrow(j, carry):
                jv = iota * 0 + j
                r = (j * d) // 128
                o = (j * d) % 128
                for k in range(d // 16):
                    vals = plsc.load_gather(blk_v.at[slot],
                                            [k * 16 + iota, jv])
                    trans_v[slot, r, pl.ds(o + k * 16, 16)] = vals
                return carry
            lax.fori_loop(0, 128, jrow, 0)

        start_in(base, 0)

        def step(i, carry):
            slot = lax.rem(i, 2)

            @pl.when(i + 1 < cnt)
            def _():
                start_in(base + i + 1, 1 - slot)
            wait_in(slot)

            @pl.when(i >= 2)
            def _():
                wait_out(slot)
            transpose_block(slot)
            start_out(base + i, slot)
            return carry

        lax.fori_loop(0, cnt, step, 0)

        @pl.when(cnt > 1)
        def _():
            wait_out(1 - lax.rem(cnt, 2))
        wait_out(lax.rem(cnt, 2))

    return pl.kernel(
        body,
        out_type=jax.ShapeDtypeStruct((v * d // 128, 128), jnp.float32),
        mesh=mesh,
        scratch_types=[
            pltpu.VMEM((2, d, 128), jnp.float32),
            pltpu.VMEM((2, rpb, 128), jnp.float32),
            pltpu.SemaphoreType.DMA((2,)),
            pltpu.SemaphoreType.DMA((2,)),
        ],
        compiler_params=pltpu.CompilerParams(use_tc_tiling_on_sc=True,
                                             needs_layout_passes=False),
    )


def _gather_kernel(n, v, d, v_lin):
    rows_per_w = n // NUM_WORKERS
    n_chunks = rows_per_w // CHUNK
    n_groups = n_chunks // NBUF
    n_tail = v - v_lin
    mesh = plsc.VectorSubcoreMesh(core_axis_name="c", subcore_axis_name="s")

    def body(idx_hbm, table_hbm, tail_hbm, out_hbm, idx_v, rows_v, tail_v,
             flags_v, *sems):
        iota = lax.iota(jnp.int32, 16)
        sem_g = sems[:NBUF]
        sem_w = sems[NBUF:]
        wid = lax.axis_index("s") * NUM_CORES + lax.axis_index("c")
        wbase = wid * rows_per_w
        t2 = table_hbm

        def start_gather(c, slot):
            pltpu.async_copy(t2.at[idx_v.at[pl.ds(c * CHUNK, CHUNK)]],
                             rows_v.at[slot], sem_g[slot])

        def wait_gather(slot):
            pltpu.make_async_copy(t2.at[idx_v.at[pl.ds(0, CHUNK)]],
                                  rows_v.at[slot], sem_g[slot]).wait()

        def start_write(c, slot):
            pltpu.async_copy(rows_v.at[slot],
                             out_hbm.at[pl.ds(wbase + c * CHUNK, CHUNK),
                                        pl.ds(0, d)],
                             sem_w[slot])

        def wait_write(slot):
            pltpu.make_async_copy(rows_v.at[slot],
                                  out_hbm.at[pl.ds(wbase, CHUNK), pl.ds(0, d)],
                                  sem_w[slot]).wait()

        def fix_tail(c, slot):
            @pl.when(flags_v[c] >= v_lin)
            def _():
                def vgroup(g, carry):
                    idxv = idx_v[pl.ds(c * CHUNK + g * 16, 16)]
                    gmax = lax.reduce_max(idxv, axes=(0,))

                    @pl.when(gmax >= v_lin)
                    def _():
                        for l in range(16):
                            ival = idxv[l]

                            @pl.when(ival >= v_lin)
                            def _():
                                trow = ival - v_lin
                                for k in range(d // 16):
                                    tv = plsc.load_gather(
                                        tail_v,
                                        [iota * 0 + trow, k * 16 + iota])
                                    rows_v[slot, g * 16 + l,
                                           pl.ds(k * 16, 16)] = tv
                    return carry
                lax.fori_loop(0, CHUNK // 16, vgroup, 0)

        pltpu.sync_copy(idx_hbm.at[pl.ds(wbase, rows_per_w)], idx_v)
        pltpu.sync_copy(tail_hbm, tail_v)

        def flag_chunk(c, carry):
            def vmax(k, m):
                return jnp.maximum(m, idx_v[pl.ds(c * CHUNK + k * 16, 16)])
            m = lax.fori_loop(0, CHUNK // 16, vmax,
                              jnp.zeros((16,), jnp.int32))
            flags_v[c] = lax.reduce_max(m, axes=(0,))
            return carry
        lax.fori_loop(0, n_chunks, flag_chunk, 0)

        start_gather(0, 0)
        start_gather(1, 1)

        def group(q, carry):
            for bslot in range(NBUF):
                c = q * NBUF + bslot
                wait_gather(bslot)
                fix_tail(c, bslot)
                start_write(c, bslot)
                nxt = (bslot + 2) % NBUF
                if bslot < 2:
                    @pl.when(q > 0)
                    def _():
                        wait_write(nxt)
                    start_gather(c + 2, nxt)
                else:
                    wait_write(nxt)

                    @pl.when(q < n_groups - 1)
                    def _():
                        start_gather(c + 2, nxt)
            return carry

        lax.fori_loop(0, n_groups, group, 0)

        wait_write(2)
        wait_write(3)

    return pl.kernel(
        body,
        out_type=jax.ShapeDtypeStruct((n, 128), jnp.float32),
        mesh=mesh,
        scratch_types=[
            pltpu.VMEM((rows_per_w,), jnp.int32),
            pltpu.VMEM((NBUF, CHUNK, d), jnp.float32),
            pltpu.VMEM((n_tail, d), jnp.float32),
            pltpu.SMEM((n_chunks,), jnp.int32),
        ] + [pltpu.SemaphoreType.DMA] * (2 * NBUF),
        compiler_params=pltpu.CompilerParams(use_tc_tiling_on_sc=False,
                                             needs_layout_passes=False),
    )


def kernel(x, emb_table):
    b, s = x.shape
    v, d = emb_table.shape
    n = b * s
    v_lin = (v // 128) * 128
    table_lin = _transpose_kernel(d, v)(emb_table.T).reshape(v, d)
    tail = lax.slice(emb_table, (v_lin, 0), (v, d))
    out128 = _gather_kernel(n, v, d, v_lin)(x.reshape(n), table_lin, tail)
    return out128[:, :d].reshape(b, s, d)
